# baseline (speedup 1.0000x reference)
import functools

import jax
import jax.numpy as jnp
from jax import lax
from jax.experimental import pallas as pl
from jax.experimental.pallas import tpu as pltpu
from jax.experimental.pallas import tpu_sc as plsc

B = 16384
H = 20
D = 64
NUM_OUT = 3

NC = 2
NS = 16
NW = NC * NS
BPW = B // NW
BPC = 4
IPC = BPC * H
NCH = BPW // BPC
DV = D // 16


def _pool_body(idx_hbm, table_hbm, out_hbm, idx_v, rows0, rows1, out_v,
               sem0, sem1):
    wid = lax.axis_index("s") * NC + lax.axis_index("c")
    pltpu.sync_copy(idx_hbm.at[pl.ds(wid * NCH, NCH)], idx_v)
    pltpu.async_copy(table_hbm.at[idx_v.at[0]], rows0, sem0)
    pltpu.async_copy(table_hbm.at[idx_v.at[1]], rows1, sem1)

    def outer(g, carry):
        for s, (rows, sem) in enumerate(((rows0, sem0), (rows1, sem1))):
            j = 2 * g + s
            pltpu.make_async_copy(table_hbm.at[idx_v.at[0]], rows, sem).wait()
            for bl in range(BPC):
                acc = [rows[bl * H, pl.ds(c * 16, 16)] for c in range(DV)]
                for l in range(1, H):
                    for c in range(DV):
                        acc[c] = acc[c] + rows[bl * H + l, pl.ds(c * 16, 16)]
                base = (j * BPC + bl) * D
                for c in range(DV):
                    out_v[pl.ds(base + c * 16, 16)] = acc[c] * (1.0 / H)

            @pl.when(j + 2 < NCH)
            def _():
                pltpu.async_copy(table_hbm.at[idx_v.at[j + 2]], rows, sem)
        return carry

    lax.fori_loop(0, NCH // 2, outer, 0)
    pltpu.sync_copy(out_v, out_hbm.at[pl.ds(wid * (BPW * D), BPW * D)])


def _pool(idx2d, table):
    f = pl.kernel(
        _pool_body,
        out_type=jax.ShapeDtypeStruct((B * D,), jnp.float32),
        mesh=plsc.VectorSubcoreMesh(core_axis_name="c", subcore_axis_name="s",
                                    num_cores=NC, num_subcores=NS),
        scratch_types=[
            pltpu.VMEM((NCH, IPC), jnp.int32),
            pltpu.VMEM((IPC, D), jnp.float32),
            pltpu.VMEM((IPC, D), jnp.float32),
            pltpu.VMEM((BPW * D,), jnp.float32),
            pltpu.SemaphoreType.DMA,
            pltpu.SemaphoreType.DMA,
        ],
        compiler_params=pltpu.CompilerParams(use_tc_tiling_on_sc=False),
    )
    return f(idx2d, table)


def _mlp_body(pooled_ref, state_ref, swt_ref, sb_ref, w1t_ref, b1_ref,
              w2t_ref, b2_ref, out_ref):
    x = pooled_ref[...] + jnp.dot(state_ref[...], swt_ref[...],
                                  preferred_element_type=jnp.float32)
    x = x + sb_ref[...]
    h = jnp.maximum(x, 0.0)
    h = jnp.dot(h, w1t_ref[...], preferred_element_type=jnp.float32)
    h = jnp.maximum(h + b1_ref[...], 0.0)
    out_ref[...] = jnp.dot(h, w2t_ref[...],
                           preferred_element_type=jnp.float32) + b2_ref[...]


def _mlp(pooled, state, swt, sb, w1t, b1, w2t, b2):
    blk = 2048
    grid = B // blk
    rep = lambda shape: pl.BlockSpec(shape, lambda i: (0, 0))
    return pl.pallas_call(
        _mlp_body,
        grid=(grid,),
        in_specs=[
            pl.BlockSpec((blk, D), lambda i: (i, 0)),
            pl.BlockSpec((blk, NUM_OUT), lambda i: (i, 0)),
            rep((NUM_OUT, D)),
            rep((1, D)),
            rep((D, D // 2)),
            rep((1, D // 2)),
            rep((D // 2, NUM_OUT)),
            rep((1, NUM_OUT)),
        ],
        out_specs=pl.BlockSpec((blk, NUM_OUT), lambda i: (i, 0)),
        out_shape=jax.ShapeDtypeStruct((B, NUM_OUT), jnp.float32),
    )(pooled, state, swt, sb, w1t, b1, w2t, b2)


def kernel(players, state, emb_table, state_W, state_b, W1, b1, W2, b2):
    idx2d = players.astype(jnp.int32).reshape(NW * NCH, IPC)
    pooled = _pool(idx2d, emb_table).reshape(B, D)
    return _mlp(pooled, state,
                state_W.T, state_b.reshape(1, D),
                W1.T, b1.reshape(1, D // 2),
                W2.T, b2.reshape(1, NUM_OUT))
